# baseline (speedup 1.0000x reference)
import jax
import jax.numpy as jnp
from jax import lax
from jax.experimental import pallas as pl
from jax.experimental.pallas import tpu as pltpu
from jax.experimental.pallas import tpu_sc as plsc

N = 10000
D = 128
HALF = D // 2
E = 160000
NEG_SLOPE = 0.2

NUM_TILES = 16
EDGES_PER_TILE = E // NUM_TILES
CHUNK = 128
EDGES_PER_TILE_PAD = 10240
CHUNKS_PER_TILE = EDGES_PER_TILE_PAD // CHUNK
NPAD = 10240
ROWS_PER_TILE = NPAD // NUM_TILES
DEG_W = 16
NSLOT = 4
PF = 2

_MESH = dict(core_axis_name="c", subcore_axis_name="s")


def _deg_body(dst_hbm, deg_out, dstv, onesv, zd, dega):
    c = lax.axis_index("c")
    s = lax.axis_index("s")
    rbase = s * ROWS_PER_TILE

    pltpu.sync_copy(dst_hbm.at[c, s], dstv)

    one16 = jnp.full((16,), 1.0, jnp.float32)
    zero16 = jnp.zeros((16,), jnp.float32)

    def fill(r, carry):
        onesv[r] = one16
        zd[r] = zero16
        return carry

    lax.fori_loop(0, CHUNK, fill, 0)
    for k in range(ROWS_PER_TILE // 128):
        pltpu.sync_copy(zd, dega.at[pl.ds(rbase + k * 128, 128)])
    plsc.subcore_barrier()

    def chunk_body(i, carry):
        pltpu.sync_copy(onesv, dega.at[dstv.at[i]], add=True)
        return carry

    lax.fori_loop(0, CHUNKS_PER_TILE, chunk_body, 0)
    plsc.subcore_barrier()
    pltpu.sync_copy(dega.at[pl.ds(rbase, ROWS_PER_TILE)],
                    deg_out.at[c, pl.ds(rbase, ROWS_PER_TILE)])


_sc_deg = pl.kernel(
    _deg_body,
    mesh=plsc.VectorSubcoreMesh(**_MESH),
    out_type=jax.ShapeDtypeStruct((2, NPAD, DEG_W), jnp.float32),
    scratch_types=[
        pltpu.VMEM((CHUNKS_PER_TILE, CHUNK), jnp.int32),
        pltpu.VMEM((CHUNK, DEG_W), jnp.float32),
        pltpu.VMEM((CHUNK, DEG_W), jnp.float32),
        pltpu.VMEM_SHARED((NPAD, DEG_W), jnp.float32),
    ],
    compiler_params=pltpu.CompilerParams(use_tc_tiling_on_sc=False),
)


def _agg_body(h0_hbm, h1_hbm, src_hbm, dst_hbm, a_out,
              srcv, dstv, rows, zbuf, acc, semg, *sems):
    c = lax.axis_index("c")
    s = lax.axis_index("s")
    rbase = s * ROWS_PER_TILE

    pltpu.sync_copy(src_hbm.at[c, s], srcv)
    pltpu.sync_copy(dst_hbm.at[c, s], dstv)

    zero16 = jnp.zeros((16,), jnp.float32)

    def fill_zero(r, carry):
        for j in range(HALF // 16):
            zbuf[r, pl.ds(j * 16, 16)] = zero16
        return carry

    lax.fori_loop(0, 128, fill_zero, 0)

    def issue_gather(h_hbm, chunk, b):
        pltpu.async_copy(h_hbm.at[srcv.at[chunk]], rows.at[b], semg)

    def wait_gather(h_hbm, chunk, b):
        pltpu.make_async_copy(h_hbm.at[srcv.at[chunk]], rows.at[b],
                              semg).wait()

    def issue_scatter(chunk, b):
        pltpu.async_copy(rows.at[b], acc.at[dstv.at[chunk]], sems[b],
                         add=True)

    def wait_scatter(b):
        pltpu.make_async_copy(rows.at[b], acc.at[dstv.at[0]],
                              sems[b]).wait()

    for half in range(2):
        h_hbm = h0_hbm if half == 0 else h1_hbm
        for k in range(ROWS_PER_TILE // 128):
            pltpu.sync_copy(zbuf, acc.at[pl.ds(rbase + k * 128, 128)])
        plsc.subcore_barrier()

        for b in range(PF):
            issue_gather(h_hbm, b, b)
        for cstep in range(NSLOT - PF):
            wait_gather(h_hbm, cstep, cstep)
            issue_scatter(cstep, cstep)
            issue_gather(h_hbm, cstep + PF, cstep + PF)

        def group_body(g, carry):
            base = (NSLOT - PF) + g * NSLOT
            for k in range(NSLOT):
                cc = base + k
                b = (NSLOT - PF + k) % NSLOT
                wait_gather(h_hbm, cc, b)
                issue_scatter(cc, b)
                b2 = k
                wait_scatter(b2)
                issue_gather(h_hbm, cc + PF, b2)
            return carry

        lax.fori_loop(0, (CHUNKS_PER_TILE - NSLOT) // NSLOT, group_body, 0)
        for cc in range(CHUNKS_PER_TILE - PF, CHUNKS_PER_TILE):
            b = cc % NSLOT
            wait_gather(h_hbm, cc, b)
            issue_scatter(cc, b)
        for b in range(NSLOT):
            wait_scatter(b)
        plsc.subcore_barrier()

        pltpu.sync_copy(acc.at[pl.ds(rbase, ROWS_PER_TILE)],
                        a_out.at[c, half, pl.ds(rbase, ROWS_PER_TILE)])


_sc_agg = pl.kernel(
    _agg_body,
    mesh=plsc.VectorSubcoreMesh(**_MESH),
    out_type=jax.ShapeDtypeStruct((2, 2, NPAD, HALF), jnp.float32),
    scratch_types=[
        pltpu.VMEM((CHUNKS_PER_TILE, CHUNK), jnp.int32),
        pltpu.VMEM((CHUNKS_PER_TILE, CHUNK), jnp.int32),
        pltpu.VMEM((NSLOT, CHUNK, HALF), jnp.float32),
        pltpu.VMEM((128, HALF), jnp.float32),
        pltpu.VMEM_SHARED((NPAD, HALF), jnp.float32),
        pltpu.SemaphoreType.DMA,
    ] + [pltpu.SemaphoreType.DMA] * NSLOT,
    compiler_params=pltpu.CompilerParams(use_tc_tiling_on_sc=False),
)

def _dense_body(split_out, a_ref, deg_ref, w0_ref, w1_ref, b_ref, *out_refs):
    d0 = deg_ref[0][:, 0:1]
    d1 = deg_ref[1][:, 0:1]
    inv0 = 1.0 / jnp.maximum(d0, 1.0)
    inv1 = 1.0 / jnp.maximum(d1, 1.0)
    h = b_ref[...]
    for half in range(2):
        ws = pl.ds(half * HALF, HALF)
        h = h + jnp.dot(a_ref[0, half] * inv0, w0_ref[ws, :],
                        preferred_element_type=jnp.float32)
        h = h + jnp.dot(a_ref[1, half] * inv1, w1_ref[ws, :],
                        preferred_element_type=jnp.float32)
    h = jnp.where(h >= 0.0, h, NEG_SLOPE * h)
    if split_out:
        out_refs[0][...] = h[:, :HALF]
        out_refs[1][...] = h[:, HALF:]
    else:
        out_refs[0][...] = h


def _dense(a, deg, w0, w1, b, split_out):
    BR = 2000
    grid = (N // BR,)
    if split_out:
        out_shape = [jax.ShapeDtypeStruct((N, HALF), jnp.float32)] * 2
        out_specs = [pl.BlockSpec((BR, HALF), lambda i: (i, 0))] * 2
    else:
        out_shape = jax.ShapeDtypeStruct((N, D), jnp.float32)
        out_specs = pl.BlockSpec((BR, D), lambda i: (i, 0))
    return pl.pallas_call(
        lambda *refs: _dense_body(split_out, *refs),
        grid=grid,
        in_specs=[
            pl.BlockSpec((2, 2, BR, HALF), lambda i: (0, 0, i, 0)),
            pl.BlockSpec((2, BR, DEG_W), lambda i: (0, i, 0)),
            pl.BlockSpec((D, D), lambda i: (0, 0)),
            pl.BlockSpec((D, D), lambda i: (0, 0)),
            pl.BlockSpec((1, D), lambda i: (0, 0)),
        ],
        out_specs=out_specs,
        out_shape=out_shape,
    )(a, deg, w0, w1, b.reshape(1, D))


@jax.jit
def kernel(x, edge_index_r0, edge_index_r1, W0_r0, W0_r1, b0, W1_r0, W1_r1, b1):
    pad = EDGES_PER_TILE_PAD - EDGES_PER_TILE
    src = jnp.stack([edge_index_r0[0], edge_index_r1[0]]) \
             .reshape(2, NUM_TILES, EDGES_PER_TILE)
    src = jnp.pad(src, ((0, 0), (0, 0), (0, pad))) \
             .reshape(2, NUM_TILES, CHUNKS_PER_TILE, CHUNK)
    dst = jnp.stack([edge_index_r0[1], edge_index_r1[1]]) \
             .reshape(2, NUM_TILES, EDGES_PER_TILE)
    dst = jnp.pad(dst, ((0, 0), (0, 0), (0, pad)), constant_values=NPAD - 1) \
             .reshape(2, NUM_TILES, CHUNKS_PER_TILE, CHUNK)

    deg = _sc_deg(dst)
    a1 = _sc_agg(x[:, :HALF], x[:, HALF:], src, dst)
    h1_lo, h1_hi = _dense(a1, deg, W0_r0, W0_r1, b0, split_out=True)
    a2 = _sc_agg(h1_lo, h1_hi, src, dst)
    h2 = _dense(a2, deg, W1_r0, W1_r1, b1, split_out=False)
    return h2


# speedup vs baseline: 1.0077x; 1.0077x over previous
import jax
import jax.numpy as jnp
from jax import lax
from jax.experimental import pallas as pl
from jax.experimental.pallas import tpu as pltpu
from jax.experimental.pallas import tpu_sc as plsc

N = 10000
D = 128
HALF = D // 2
E = 160000
NEG_SLOPE = 0.2

NUM_TILES = 16
EDGES_PER_TILE = E // NUM_TILES
CHUNK = 256
EDGES_PER_TILE_PAD = 10240
CHUNKS_PER_TILE = EDGES_PER_TILE_PAD // CHUNK
NPAD = 10240
ROWS_PER_TILE = NPAD // NUM_TILES
DEG_W = 16
NBUF = 2

_MESH = dict(core_axis_name="c", subcore_axis_name="s")


def _deg_body(dst_hbm, deg_out, dstv, onesv, zd, dega):
    c = lax.axis_index("c")
    s = lax.axis_index("s")
    rbase = s * ROWS_PER_TILE

    pltpu.sync_copy(dst_hbm.at[c, s], dstv)

    one16 = jnp.full((16,), 1.0, jnp.float32)
    zero16 = jnp.zeros((16,), jnp.float32)

    def fill(r, carry):
        onesv[r] = one16
        return carry

    lax.fori_loop(0, CHUNK, fill, 0)

    def fill_z(r, carry):
        zd[r] = zero16
        return carry

    lax.fori_loop(0, 128, fill_z, 0)
    for k in range(ROWS_PER_TILE // 128):
        pltpu.sync_copy(zd, dega.at[pl.ds(rbase + k * 128, 128)])
    plsc.subcore_barrier()

    def chunk_body(i, carry):
        pltpu.sync_copy(onesv, dega.at[dstv.at[i]], add=True)
        return carry

    lax.fori_loop(0, CHUNKS_PER_TILE, chunk_body, 0)
    plsc.subcore_barrier()
    pltpu.sync_copy(dega.at[pl.ds(rbase, ROWS_PER_TILE)],
                    deg_out.at[c, pl.ds(rbase, ROWS_PER_TILE)])


_sc_deg = pl.kernel(
    _deg_body,
    mesh=plsc.VectorSubcoreMesh(**_MESH),
    out_type=jax.ShapeDtypeStruct((2, NPAD, DEG_W), jnp.float32),
    scratch_types=[
        pltpu.VMEM((CHUNKS_PER_TILE, CHUNK), jnp.int32),
        pltpu.VMEM((CHUNK, DEG_W), jnp.float32),
        pltpu.VMEM((128, DEG_W), jnp.float32),
        pltpu.VMEM_SHARED((NPAD, DEG_W), jnp.float32),
    ],
    compiler_params=pltpu.CompilerParams(use_tc_tiling_on_sc=False),
)


def _agg_body(h0_hbm, h1_hbm, src_hbm, dst_hbm, a_out,
              srcv, dstv, rows, zbuf, acc, sem):
    c = lax.axis_index("c")
    s = lax.axis_index("s")
    rbase = s * ROWS_PER_TILE

    pltpu.sync_copy(src_hbm.at[c, s], srcv)
    pltpu.sync_copy(dst_hbm.at[c, s], dstv)

    zero16 = jnp.zeros((16,), jnp.float32)

    def fill_zero(r, carry):
        for j in range(HALF // 16):
            zbuf[r, pl.ds(j * 16, 16)] = zero16
        return carry

    lax.fori_loop(0, 128, fill_zero, 0)

    for half in range(2):
        h_hbm = h0_hbm if half == 0 else h1_hbm
        for k in range(ROWS_PER_TILE // 128):
            pltpu.sync_copy(zbuf, acc.at[pl.ds(rbase + k * 128, 128)])
        plsc.subcore_barrier()

        for b in range(NBUF):
            pltpu.async_copy(h_hbm.at[srcv.at[b]], rows.at[b], sem)

        def group_body(g, carry):
            for b in range(NBUF):
                chunk = g * NBUF + b
                pltpu.make_async_copy(
                    h_hbm.at[srcv.at[chunk]], rows.at[b], sem).wait()
                pltpu.sync_copy(rows.at[b], acc.at[dstv.at[chunk]], add=True)
                pltpu.async_copy(
                    h_hbm.at[srcv.at[chunk + NBUF]], rows.at[b], sem)
            return carry

        lax.fori_loop(0, CHUNKS_PER_TILE // NBUF - 1, group_body, 0)
        for b in range(NBUF):
            chunk = CHUNKS_PER_TILE - NBUF + b
            pltpu.make_async_copy(
                h_hbm.at[srcv.at[chunk]], rows.at[b], sem).wait()
            pltpu.sync_copy(rows.at[b], acc.at[dstv.at[chunk]], add=True)
        plsc.subcore_barrier()

        pltpu.sync_copy(acc.at[pl.ds(rbase, ROWS_PER_TILE)],
                        a_out.at[c, half, pl.ds(rbase, ROWS_PER_TILE)])


_sc_agg = pl.kernel(
    _agg_body,
    mesh=plsc.VectorSubcoreMesh(**_MESH),
    out_type=jax.ShapeDtypeStruct((2, 2, NPAD, HALF), jnp.float32),
    scratch_types=[
        pltpu.VMEM((CHUNKS_PER_TILE, CHUNK), jnp.int32),
        pltpu.VMEM((CHUNKS_PER_TILE, CHUNK), jnp.int32),
        pltpu.VMEM((NBUF, CHUNK, HALF), jnp.float32),
        pltpu.VMEM((128, HALF), jnp.float32),
        pltpu.VMEM_SHARED((NPAD, HALF), jnp.float32),
        pltpu.SemaphoreType.DMA,
    ],
    compiler_params=pltpu.CompilerParams(use_tc_tiling_on_sc=False),
)

def _dense_body(split_out, a_ref, deg_ref, w0_ref, w1_ref, b_ref, *out_refs):
    d0 = deg_ref[0][:, 0:1]
    d1 = deg_ref[1][:, 0:1]
    inv0 = 1.0 / jnp.maximum(d0, 1.0)
    inv1 = 1.0 / jnp.maximum(d1, 1.0)
    h = b_ref[...]
    for half in range(2):
        ws = pl.ds(half * HALF, HALF)
        h = h + jnp.dot(a_ref[0, half] * inv0, w0_ref[ws, :],
                        preferred_element_type=jnp.float32)
        h = h + jnp.dot(a_ref[1, half] * inv1, w1_ref[ws, :],
                        preferred_element_type=jnp.float32)
    h = jnp.where(h >= 0.0, h, NEG_SLOPE * h)
    if split_out:
        out_refs[0][...] = h[:, :HALF]
        out_refs[1][...] = h[:, HALF:]
    else:
        out_refs[0][...] = h


def _dense(a, deg, w0, w1, b, split_out):
    BR = 2000
    grid = (N // BR,)
    if split_out:
        out_shape = [jax.ShapeDtypeStruct((N, HALF), jnp.float32)] * 2
        out_specs = [pl.BlockSpec((BR, HALF), lambda i: (i, 0))] * 2
    else:
        out_shape = jax.ShapeDtypeStruct((N, D), jnp.float32)
        out_specs = pl.BlockSpec((BR, D), lambda i: (i, 0))
    return pl.pallas_call(
        lambda *refs: _dense_body(split_out, *refs),
        grid=grid,
        in_specs=[
            pl.BlockSpec((2, 2, BR, HALF), lambda i: (0, 0, i, 0)),
            pl.BlockSpec((2, BR, DEG_W), lambda i: (0, i, 0)),
            pl.BlockSpec((D, D), lambda i: (0, 0)),
            pl.BlockSpec((D, D), lambda i: (0, 0)),
            pl.BlockSpec((1, D), lambda i: (0, 0)),
        ],
        out_specs=out_specs,
        out_shape=out_shape,
    )(a, deg, w0, w1, b.reshape(1, D))


@jax.jit
def kernel(x, edge_index_r0, edge_index_r1, W0_r0, W0_r1, b0, W1_r0, W1_r1, b1):
    pad = EDGES_PER_TILE_PAD - EDGES_PER_TILE
    src = jnp.stack([edge_index_r0[0], edge_index_r1[0]]) \
             .reshape(2, NUM_TILES, EDGES_PER_TILE)
    src = jnp.pad(src, ((0, 0), (0, 0), (0, pad))) \
             .reshape(2, NUM_TILES, CHUNKS_PER_TILE, CHUNK)
    dst = jnp.stack([edge_index_r0[1], edge_index_r1[1]]) \
             .reshape(2, NUM_TILES, EDGES_PER_TILE)
    dst = jnp.pad(dst, ((0, 0), (0, 0), (0, pad)), constant_values=NPAD - 1) \
             .reshape(2, NUM_TILES, CHUNKS_PER_TILE, CHUNK)

    deg = _sc_deg(dst)
    a1 = _sc_agg(x[:, :HALF], x[:, HALF:], src, dst)
    h1_lo, h1_hi = _dense(a1, deg, W0_r0, W0_r1, b0, split_out=True)
    a2 = _sc_agg(h1_lo, h1_hi, src, dst)
    h2 = _dense(a2, deg, W1_r0, W1_r1, b1, split_out=False)
    return h2


# speedup vs baseline: 1.8076x; 1.7937x over previous
import jax
import jax.numpy as jnp
from jax import lax
from jax.experimental import pallas as pl
from jax.experimental.pallas import tpu as pltpu
from jax.experimental.pallas import tpu_sc as plsc

N = 10000
D = 128
E = 160000
NEG_SLOPE = 0.2

NUM_TILES = 16
EDGES_PER_TILE = E // NUM_TILES
CHUNK = 128
EDGES_PER_TILE_PAD = 10240
CHUNKS_PER_TILE = EDGES_PER_TILE_PAD // CHUNK
NPAD = 10240
ROWS_PER_TILE = NPAD // NUM_TILES
DEG_W = 16
NBUF = 4

_MESH = dict(core_axis_name="c", subcore_axis_name="s")


def _deg_body(dst_hbm, deg_out, dstv, onesv, zd, dega):
    c = lax.axis_index("c")
    s = lax.axis_index("s")
    rbase = s * ROWS_PER_TILE

    pltpu.sync_copy(dst_hbm.at[c, s], dstv)

    one16 = jnp.full((16,), 1.0, jnp.float32)
    zero16 = jnp.zeros((16,), jnp.float32)

    def fill(r, carry):
        onesv[r] = one16
        return carry

    lax.fori_loop(0, CHUNK, fill, 0)

    def fill_z(r, carry):
        zd[r] = zero16
        return carry

    lax.fori_loop(0, 128, fill_z, 0)
    for k in range(ROWS_PER_TILE // 128):
        pltpu.sync_copy(zd, dega.at[pl.ds(rbase + k * 128, 128)])
    plsc.subcore_barrier()

    def chunk_body(i, carry):
        pltpu.sync_copy(onesv, dega.at[dstv.at[i]], add=True)
        return carry

    lax.fori_loop(0, CHUNKS_PER_TILE, chunk_body, 0)
    plsc.subcore_barrier()
    pltpu.sync_copy(dega.at[pl.ds(rbase, ROWS_PER_TILE)],
                    deg_out.at[c, pl.ds(rbase, ROWS_PER_TILE)])


_sc_deg = pl.kernel(
    _deg_body,
    mesh=plsc.VectorSubcoreMesh(**_MESH),
    out_type=jax.ShapeDtypeStruct((2, NPAD, DEG_W), jnp.float32),
    scratch_types=[
        pltpu.VMEM((CHUNKS_PER_TILE, CHUNK), jnp.int32),
        pltpu.VMEM((CHUNK, DEG_W), jnp.float32),
        pltpu.VMEM((128, DEG_W), jnp.float32),
        pltpu.VMEM_SHARED((NPAD, DEG_W), jnp.float32),
    ],
    compiler_params=pltpu.CompilerParams(use_tc_tiling_on_sc=False),
)


def _agg_body(h_hbm, src_hbm, dst_hbm, a_out,
              srcv, dstv, rows, zbuf, acc, sem):
    c = lax.axis_index("c")
    s = lax.axis_index("s")
    rbase = s * ROWS_PER_TILE

    pltpu.sync_copy(src_hbm.at[c, s], srcv)
    pltpu.sync_copy(dst_hbm.at[c, s], dstv)

    zero32 = jnp.zeros((32,), jnp.bfloat16)

    def fill_zero(r, carry):
        for j in range(D // 32):
            zbuf[r, pl.ds(j * 32, 32)] = zero32
        return carry

    lax.fori_loop(0, 128, fill_zero, 0)

    for k in range(ROWS_PER_TILE // 128):
        pltpu.sync_copy(zbuf, acc.at[pl.ds(rbase + k * 128, 128)])
    plsc.subcore_barrier()

    for b in range(NBUF):
        pltpu.async_copy(h_hbm.at[srcv.at[b]], rows.at[b], sem)

    def group_body(g, carry):
        for b in range(NBUF):
            chunk = g * NBUF + b
            pltpu.make_async_copy(
                h_hbm.at[srcv.at[chunk]], rows.at[b], sem).wait()
            pltpu.sync_copy(rows.at[b], acc.at[dstv.at[chunk]], add=True)
            pltpu.async_copy(
                h_hbm.at[srcv.at[chunk + NBUF]], rows.at[b], sem)
        return carry

    lax.fori_loop(0, CHUNKS_PER_TILE // NBUF - 1, group_body, 0)
    for b in range(NBUF):
        chunk = CHUNKS_PER_TILE - NBUF + b
        pltpu.make_async_copy(
            h_hbm.at[srcv.at[chunk]], rows.at[b], sem).wait()
        pltpu.sync_copy(rows.at[b], acc.at[dstv.at[chunk]], add=True)
    plsc.subcore_barrier()

    pltpu.sync_copy(acc.at[pl.ds(rbase, ROWS_PER_TILE)],
                    a_out.at[c, pl.ds(rbase, ROWS_PER_TILE)])


_sc_agg = pl.kernel(
    _agg_body,
    mesh=plsc.VectorSubcoreMesh(**_MESH),
    out_type=jax.ShapeDtypeStruct((2, NPAD, D), jnp.bfloat16),
    scratch_types=[
        pltpu.VMEM((CHUNKS_PER_TILE, CHUNK), jnp.int32),
        pltpu.VMEM((CHUNKS_PER_TILE, CHUNK), jnp.int32),
        pltpu.VMEM((NBUF, CHUNK, D), jnp.bfloat16),
        pltpu.VMEM((128, D), jnp.bfloat16),
        pltpu.VMEM_SHARED((NPAD, D), jnp.bfloat16),
        pltpu.SemaphoreType.DMA,
    ],
    compiler_params=pltpu.CompilerParams(use_tc_tiling_on_sc=False),
)


def _dense_body(out_bf16, a_ref, deg_ref, w0_ref, w1_ref, b_ref, o_ref):
    d0 = deg_ref[0][:, 0:1]
    d1 = deg_ref[1][:, 0:1]
    inv0 = 1.0 / jnp.maximum(d0, 1.0)
    inv1 = 1.0 / jnp.maximum(d1, 1.0)
    a0 = a_ref[0].astype(jnp.float32) * inv0
    a1 = a_ref[1].astype(jnp.float32) * inv1
    h = (jnp.dot(a0, w0_ref[...], preferred_element_type=jnp.float32)
         + jnp.dot(a1, w1_ref[...], preferred_element_type=jnp.float32)
         + b_ref[...])
    h = jnp.where(h >= 0.0, h, NEG_SLOPE * h)
    if out_bf16:
        o_ref[...] = h.astype(jnp.bfloat16)
    else:
        o_ref[...] = h


def _dense(a, deg, w0, w1, b, out_bf16):
    BR = 2000
    grid = (N // BR,)
    odt = jnp.bfloat16 if out_bf16 else jnp.float32
    return pl.pallas_call(
        lambda *refs: _dense_body(out_bf16, *refs),
        grid=grid,
        in_specs=[
            pl.BlockSpec((2, BR, D), lambda i: (0, i, 0)),
            pl.BlockSpec((2, BR, DEG_W), lambda i: (0, i, 0)),
            pl.BlockSpec((D, D), lambda i: (0, 0)),
            pl.BlockSpec((D, D), lambda i: (0, 0)),
            pl.BlockSpec((1, D), lambda i: (0, 0)),
        ],
        out_specs=pl.BlockSpec((BR, D), lambda i: (i, 0)),
        out_shape=jax.ShapeDtypeStruct((N, D), odt),
    )(a, deg, w0, w1, b.reshape(1, D))


@jax.jit
def kernel(x, edge_index_r0, edge_index_r1, W0_r0, W0_r1, b0, W1_r0, W1_r1, b1):
    pad = EDGES_PER_TILE_PAD - EDGES_PER_TILE
    src = jnp.stack([edge_index_r0[0], edge_index_r1[0]]) \
             .reshape(2, NUM_TILES, EDGES_PER_TILE)
    src = jnp.pad(src, ((0, 0), (0, 0), (0, pad))) \
             .reshape(2, NUM_TILES, CHUNKS_PER_TILE, CHUNK)
    dst = jnp.stack([edge_index_r0[1], edge_index_r1[1]]) \
             .reshape(2, NUM_TILES, EDGES_PER_TILE)
    dst = jnp.pad(dst, ((0, 0), (0, 0), (0, pad)), constant_values=NPAD - 1) \
             .reshape(2, NUM_TILES, CHUNKS_PER_TILE, CHUNK)

    deg = _sc_deg(dst)
    a1 = _sc_agg(x.astype(jnp.bfloat16), src, dst)
    h1 = _dense(a1, deg, W0_r0, W0_r1, b0, out_bf16=True)
    a2 = _sc_agg(h1, src, dst)
    h2 = _dense(a2, deg, W1_r0, W1_r1, b1, out_bf16=False)
    return h2
